# baseline (speedup 1.0000x reference)
import functools

import jax
import jax.numpy as jnp
from jax.experimental import pallas as pl
from jax.experimental.pallas import tpu as pltpu
from jax.experimental.pallas import tpu_sc as plsc

_N_HASHES = 8
_BLK = 128
_LSH_R = 1.0
_W = 128
_ROW = 128


def _xbox_plus(queries, keys):
    q_norm_sq = jnp.sum(queries ** 2, axis=-1, keepdims=True)
    k_norm_sq = jnp.sum(keys ** 2, axis=-1, keepdims=True)
    MQ_sq = jnp.max(q_norm_sq, axis=1, keepdims=True)
    MK_sq = jnp.max(k_norm_sq, axis=1, keepdims=True)
    q_ext = jnp.sqrt(jnp.maximum(MQ_sq - q_norm_sq, 0.0))
    k_ext = jnp.sqrt(jnp.maximum(MK_sq - k_norm_sq, 0.0))
    Q = jnp.concatenate([queries, jnp.zeros_like(q_ext), q_ext], axis=-1)
    K = jnp.concatenate([keys, k_ext, jnp.zeros_like(k_ext)], axis=-1)
    return Q, K


def _lsh_positions(X, n_hashes, bs, r, rng):
    hb, seqlen, dim = X.shape
    ka, kb = jax.random.split(rng)
    alpha = jax.random.normal(ka, (n_hashes, dim), dtype=X.dtype)
    beta = jax.random.uniform(kb, (n_hashes, 1, 1), dtype=X.dtype) * r
    Xh = X.reshape(n_hashes, bs, seqlen, dim)
    proj = jnp.einsum('hbsd,hd->hbs', Xh, alpha)
    hashes = (proj + beta) / r
    return jnp.argsort(hashes, axis=-1).reshape(n_hashes * bs, seqlen)


def _sc_mesh():
    return plsc.VectorSubcoreMesh(core_axis_name="c", subcore_axis_name="s")


def _sc_gather2(qtab, kvtab, qidx, kidx):
    B = qidx.shape[1]

    @functools.partial(
        pl.kernel,
        mesh=_sc_mesh(),
        out_type=[
            jax.ShapeDtypeStruct((B, _ROW), qtab.dtype),
            jax.ShapeDtypeStruct((B, _ROW), kvtab.dtype),
        ],
    )
    def run(q_hbm, kv_hbm, qi_hbm, ki_hbm, sq_hbm, skv_hbm):
        def body(qi_v, ki_v, sq_v, skv_v):
            pltpu.sync_copy(q_hbm.at[qi_v.at[0]], sq_v)
            pltpu.sync_copy(kv_hbm.at[ki_v.at[0]], skv_v)

        pltpu.emit_pipeline(
            body,
            grid=(B // _W,),
            in_specs=[
                pl.BlockSpec((1, _W), lambda i: (0, i)),
                pl.BlockSpec((1, _W), lambda i: (0, i)),
            ],
            out_specs=[
                pl.BlockSpec((_W, _ROW), lambda i: (i, 0)),
                pl.BlockSpec((_W, _ROW), lambda i: (i, 0)),
            ],
            core_axis_name=("c", "s"),
            dimension_semantics=(pltpu.PARALLEL,),
        )(qi_hbm, ki_hbm, sq_hbm, skv_hbm)

    return run(qtab, kvtab, qidx, kidx)


def _sc_scatter(rows, sidx, n_rows):
    B = sidx.shape[1]

    @functools.partial(
        pl.kernel,
        mesh=_sc_mesh(),
        out_type=jax.ShapeDtypeStruct((n_rows, _ROW), rows.dtype),
    )
    def run(rows_hbm, si_hbm, o_hbm):
        def body(rows_v, si_v):
            pltpu.sync_copy(rows_v, o_hbm.at[si_v.at[0]])

        pltpu.emit_pipeline(
            body,
            grid=(B // _W,),
            in_specs=[
                pl.BlockSpec((_W, _ROW), lambda i: (i, 0)),
                pl.BlockSpec((1, _W), lambda i: (0, i)),
            ],
            out_specs=[],
            core_axis_name=("c", "s"),
            dimension_semantics=(pltpu.PARALLEL,),
        )(rows_hbm, si_hbm)

    return run(rows, sidx)


def _attn_blocks_kernel(dim, v_dim, sq_ref, skv_ref, out_ref):
    q = sq_ref[0][..., :dim]
    k = skv_ref[0][..., :dim]
    v = skv_ref[0][..., dim:dim + v_dim]
    inner = jax.lax.dot_general(
        q, k, (((2,), (2,)), ((0,), (0,))),
        preferred_element_type=jnp.float32)
    m = jnp.max(inner, axis=-1, keepdims=True)
    e = jnp.exp(inner - m)
    s = jnp.sum(e, axis=-1, keepdims=True)
    bo = jax.lax.dot_general(
        e / s, v, (((2,), (1,)), ((0,), (0,))),
        preferred_element_type=jnp.float32)
    lse = m + jnp.log(s)
    lse_hi = lse.astype(jnp.bfloat16)
    lse_lo = (lse - lse_hi.astype(jnp.float32)).astype(jnp.bfloat16)
    pad = jnp.zeros(lse.shape[:-1] + (_ROW - v_dim - 2,), jnp.bfloat16)
    out_ref[0] = jnp.concatenate(
        [bo.astype(jnp.bfloat16), lse_hi, lse_lo, pad], axis=-1)


def _block_attention(sq, skv, n_steps, nblk_per_step, dim, v_dim):
    sq = sq.reshape(n_steps, nblk_per_step, _BLK, _ROW)
    skv = skv.reshape(n_steps, nblk_per_step, _BLK, _ROW)
    spec = pl.BlockSpec((1, nblk_per_step, _BLK, _ROW),
                        lambda i: (i, 0, 0, 0))
    out = pl.pallas_call(
        functools.partial(_attn_blocks_kernel, dim, v_dim),
        grid=(n_steps,),
        in_specs=[spec, spec],
        out_specs=spec,
        out_shape=jax.ShapeDtypeStruct(
            (n_steps, nblk_per_step, _BLK, _ROW), jnp.bfloat16),
    )(sq, skv)
    return out


def _combine_kernel(v_dim, n_hashes, *refs):
    nch = (len(refs) - 1) // 2
    bo_refs, qp_refs, out_ref = refs[:nch], refs[nch:2 * nch], refs[-1]
    seqlen = bo_refs[0].shape[2]
    iota = jax.lax.broadcasted_iota(jnp.int32, (seqlen, seqlen), 1)
    outs = []
    logits = []
    for cref, qref in zip(bo_refs, qp_refs):
        hc = cref.shape[0]
        for h in range(hc):
            bo = cref[h, 0]
            idx = qref[0, h]
            onehot = (idx[:, None] == iota).astype(jnp.bfloat16)
            x = jax.lax.dot_general(
                onehot, bo, (((0,), (0,)), ((), ())),
                preferred_element_type=jnp.float32)
            outs.append(x[:, :v_dim])
            logits.append(x[:, v_dim] + x[:, v_dim + 1])
    lg = jnp.stack(logits, axis=0)
    m = jnp.max(lg, axis=0)
    w = jnp.exp(lg - m[None, :])
    den = jnp.sum(w, axis=0)
    num = jnp.zeros((seqlen, v_dim), jnp.float32)
    for h in range(len(outs)):
        num = num + w[h][:, None] * outs[h]
    out_ref[0] = num / den[:, None]


def _combine(bo_chunks, qp_chunks, bs, seqlen, v_dim):
    hc = bo_chunks[0].shape[0]
    out = pl.pallas_call(
        functools.partial(_combine_kernel, v_dim, _N_HASHES),
        grid=(bs,),
        in_specs=(
            [pl.BlockSpec((hc, 1, seqlen, _ROW), lambda i: (0, i, 0, 0))
             for _ in bo_chunks]
            + [pl.BlockSpec((1, hc, seqlen), lambda i: (i, 0, 0))
               for _ in qp_chunks]
        ),
        out_specs=pl.BlockSpec((1, seqlen, v_dim), lambda i: (i, 0, 0)),
        out_shape=jax.ShapeDtypeStruct((bs, seqlen, v_dim), jnp.float32),
    )(*bo_chunks, *qp_chunks)
    return out


_N_CHUNKS = 2


def _hash_values(X, n_hashes, bs, r, rng):
    hb, seqlen, dim = X.shape
    ka, kb = jax.random.split(rng)
    alpha = jax.random.normal(ka, (n_hashes, dim), dtype=X.dtype)
    beta = jax.random.uniform(kb, (n_hashes, 1, 1), dtype=X.dtype) * r
    Xh = X.reshape(n_hashes, bs, seqlen, dim)
    proj = jnp.einsum('hbsd,hd->hbs', Xh, alpha)
    return (proj + beta) / r


def _bitonic_argsort_kernel(h_ref, idx_ref):
    v = h_ref[...]
    ix = jax.lax.broadcasted_iota(jnp.int32, v.shape, 1)
    rb, n = v.shape
    pos = jax.lax.broadcasted_iota(jnp.int32, v.shape, 1)
    k = 2
    while k <= n:
        j = k // 2
        asc = (pos & k) == 0
        while j >= 1:
            bitc = (pos & j) == 0
            vm = jnp.roll(v, -j, axis=1)
            vp = jnp.roll(v, j, axis=1)
            im = jnp.roll(ix, -j, axis=1)
            ip = jnp.roll(ix, j, axis=1)
            pv = jnp.where(bitc, vm, vp)
            pi = jnp.where(bitc, im, ip)
            less = (v < pv) | ((v == pv) & (ix < pi))
            keep = less ^ bitc ^ asc
            v = jnp.where(keep, v, pv)
            ix = jnp.where(keep, ix, pi)
            j //= 2
        k *= 2
    idx_ref[...] = ix


def _tc_argsort(h, rows_per_block=32):
    shape = h.shape
    n = shape[-1]
    rows = 1
    for s in shape[:-1]:
        rows *= s
    rows_per_block = min(rows_per_block, rows)
    h2 = h.reshape(rows, n)
    idx = pl.pallas_call(
        _bitonic_argsort_kernel,
        grid=(rows // rows_per_block,),
        in_specs=[pl.BlockSpec((rows_per_block, n), lambda i: (i, 0))],
        out_specs=pl.BlockSpec((rows_per_block, n), lambda i: (i, 0)),
        out_shape=jax.ShapeDtypeStruct((rows, n), jnp.int32),
    )(h2)
    return idx.reshape(shape)


def kernel(queries, keys, values):
    n_hashes = _N_HASHES
    bs, q_seqlen, dim = queries.shape
    k_seqlen = keys.shape[1]
    v_dim = values.shape[-1]

    Qx, Kx = _xbox_plus(queries, keys)
    Qx = jnp.tile(Qx, (n_hashes, 1, 1))
    Kx = jnp.tile(Kx, (n_hashes, 1, 1))
    rng = jax.random.key(42)
    rq, rk = jax.random.split(rng)
    q_hashes = _hash_values(Qx, n_hashes, bs, _LSH_R, rq)
    k_hashes = _hash_values(Kx, n_hashes, bs, _LSH_R, rk)

    boff = (jnp.arange(bs, dtype=jnp.int32) * q_seqlen)[None, :, None]

    qtab = jnp.concatenate(
        [queries, jnp.zeros((bs, q_seqlen, _ROW - dim), queries.dtype)],
        axis=-1).reshape(bs * q_seqlen, _ROW)
    kvtab = jnp.concatenate([keys, values], axis=-1).reshape(
        bs * k_seqlen, _ROW)

    hc = n_hashes // _N_CHUNKS
    Bc = hc * bs * q_seqlen
    nblk = q_seqlen // _BLK
    bo_chunks = []
    qp_chunks = []
    for c in range(_N_CHUNKS):
        qp = _tc_argsort(q_hashes[c * hc:(c + 1) * hc])
        kp = _tc_argsort(k_hashes[c * hc:(c + 1) * hc])
        q_abs = qp.astype(jnp.int32) + boff
        k_abs = kp.astype(jnp.int32) + boff
        sq, skv = _sc_gather2(qtab, kvtab, q_abs.reshape(1, Bc),
                              k_abs.reshape(1, Bc))
        bo_ext = _block_attention(sq, skv, hc * bs, nblk, dim, v_dim)
        bo_chunks.append(bo_ext.reshape(hc, bs, q_seqlen, _ROW))
        qp_chunks.append(
            jnp.transpose(qp.astype(jnp.int32), (1, 0, 2)))

    out = _combine(bo_chunks, qp_chunks, bs, q_seqlen, v_dim)
    return out


# speedup vs baseline: 1.8072x; 1.8072x over previous
import functools

import jax
import jax.numpy as jnp
from jax.experimental import pallas as pl
from jax.experimental.pallas import tpu as pltpu
from jax.experimental.pallas import tpu_sc as plsc

_N_HASHES = 8
_BLK = 128
_LSH_R = 1.0
_W = 128
_ROW = 128


def _xbox_plus(queries, keys):
    q_norm_sq = jnp.sum(queries ** 2, axis=-1, keepdims=True)
    k_norm_sq = jnp.sum(keys ** 2, axis=-1, keepdims=True)
    MQ_sq = jnp.max(q_norm_sq, axis=1, keepdims=True)
    MK_sq = jnp.max(k_norm_sq, axis=1, keepdims=True)
    q_ext = jnp.sqrt(jnp.maximum(MQ_sq - q_norm_sq, 0.0))
    k_ext = jnp.sqrt(jnp.maximum(MK_sq - k_norm_sq, 0.0))
    Q = jnp.concatenate([queries, jnp.zeros_like(q_ext), q_ext], axis=-1)
    K = jnp.concatenate([keys, k_ext, jnp.zeros_like(k_ext)], axis=-1)
    return Q, K


def _lsh_positions(X, n_hashes, bs, r, rng):
    hb, seqlen, dim = X.shape
    ka, kb = jax.random.split(rng)
    alpha = jax.random.normal(ka, (n_hashes, dim), dtype=X.dtype)
    beta = jax.random.uniform(kb, (n_hashes, 1, 1), dtype=X.dtype) * r
    Xh = X.reshape(n_hashes, bs, seqlen, dim)
    proj = jnp.einsum('hbsd,hd->hbs', Xh, alpha)
    hashes = (proj + beta) / r
    return jnp.argsort(hashes, axis=-1).reshape(n_hashes * bs, seqlen)


def _sc_mesh():
    return plsc.VectorSubcoreMesh(core_axis_name="c", subcore_axis_name="s")


def _sc_gather2(qtab, kvtab, qidx, kidx):
    B = qidx.shape[1]

    @functools.partial(
        pl.kernel,
        mesh=_sc_mesh(),
        out_type=[
            jax.ShapeDtypeStruct((B, _ROW), qtab.dtype),
            jax.ShapeDtypeStruct((B, _ROW), kvtab.dtype),
        ],
        scratch_types=[pltpu.SemaphoreType.DMA, pltpu.SemaphoreType.DMA],
    )
    def run(q_hbm, kv_hbm, qi_hbm, ki_hbm, sq_hbm, skv_hbm, sem1, sem2):
        def body(qi_v, ki_v, sq_v, skv_v):
            h1 = pltpu.async_copy(q_hbm.at[qi_v.at[0]], sq_v, sem1)
            h2 = pltpu.async_copy(kv_hbm.at[ki_v.at[0]], skv_v, sem2)
            h1.wait()
            h2.wait()

        pltpu.emit_pipeline(
            body,
            grid=(B // _W,),
            in_specs=[
                pl.BlockSpec((1, _W), lambda i: (0, i)),
                pl.BlockSpec((1, _W), lambda i: (0, i)),
            ],
            out_specs=[
                pl.BlockSpec((_W, _ROW), lambda i: (i, 0)),
                pl.BlockSpec((_W, _ROW), lambda i: (i, 0)),
            ],
            core_axis_name=("c", "s"),
            dimension_semantics=(pltpu.PARALLEL,),
        )(qi_hbm, ki_hbm, sq_hbm, skv_hbm)

    return run(qtab, kvtab, qidx, kidx)


def _sc_scatter(rows, sidx, n_rows):
    B = sidx.shape[1]

    @functools.partial(
        pl.kernel,
        mesh=_sc_mesh(),
        out_type=jax.ShapeDtypeStruct((n_rows, _ROW), rows.dtype),
    )
    def run(rows_hbm, si_hbm, o_hbm):
        def body(rows_v, si_v):
            pltpu.sync_copy(rows_v, o_hbm.at[si_v.at[0]])

        pltpu.emit_pipeline(
            body,
            grid=(B // _W,),
            in_specs=[
                pl.BlockSpec((_W, _ROW), lambda i: (i, 0)),
                pl.BlockSpec((1, _W), lambda i: (0, i)),
            ],
            out_specs=[],
            core_axis_name=("c", "s"),
            dimension_semantics=(pltpu.PARALLEL,),
        )(rows_hbm, si_hbm)

    return run(rows, sidx)


def _attn_blocks_kernel(dim, v_dim, sq_ref, skv_ref, out_ref):
    q = sq_ref[0][..., :dim]
    k = skv_ref[0][..., :dim]
    v = skv_ref[0][..., dim:dim + v_dim]
    inner = jax.lax.dot_general(
        q, k, (((2,), (2,)), ((0,), (0,))),
        preferred_element_type=jnp.float32)
    m = jnp.max(inner, axis=-1, keepdims=True)
    e = jnp.exp(inner - m)
    s = jnp.sum(e, axis=-1, keepdims=True)
    bo = jax.lax.dot_general(
        e / s, v, (((2,), (1,)), ((0,), (0,))),
        preferred_element_type=jnp.float32)
    lse = m + jnp.log(s)
    lse_b = jnp.broadcast_to(lse, lse.shape[:-1] + (_ROW - v_dim,))
    out_ref[0] = jnp.concatenate([bo, lse_b], axis=-1)


def _block_attention(sq, skv, n_steps, nblk_per_step, dim, v_dim):
    sq = sq.reshape(n_steps, nblk_per_step, _BLK, _ROW)
    skv = skv.reshape(n_steps, nblk_per_step, _BLK, _ROW)
    spec = pl.BlockSpec((1, nblk_per_step, _BLK, _ROW),
                        lambda i: (i, 0, 0, 0))
    out = pl.pallas_call(
        functools.partial(_attn_blocks_kernel, dim, v_dim),
        grid=(n_steps,),
        in_specs=[spec, spec],
        out_specs=spec,
        out_shape=jax.ShapeDtypeStruct(
            (n_steps, nblk_per_step, _BLK, _ROW), jnp.float32),
    )(sq, skv)
    return out


def _combine_kernel(v_dim, *refs):
    x_refs, out_ref = refs[:-1], refs[-1]
    x = jnp.concatenate([r[:, 0] for r in x_refs], axis=0)
    o = x[..., :v_dim]
    logits = x[..., v_dim]
    m = jnp.max(logits, axis=0)
    w = jnp.exp(logits - m[None, :])
    den = jnp.sum(w, axis=0)
    num = jnp.sum(w[..., None] * o, axis=0)
    out_ref[0] = num / den[:, None]


def _combine(o_uns_chunks, bs, seqlen, v_dim):
    hc = o_uns_chunks[0].shape[0]
    out = pl.pallas_call(
        functools.partial(_combine_kernel, v_dim),
        grid=(bs,),
        in_specs=[
            pl.BlockSpec((hc, 1, seqlen, _ROW), lambda i: (0, i, 0, 0))
            for _ in o_uns_chunks
        ],
        out_specs=pl.BlockSpec((1, seqlen, v_dim), lambda i: (i, 0, 0)),
        out_shape=jax.ShapeDtypeStruct((bs, seqlen, v_dim), jnp.float32),
    )(*o_uns_chunks)
    return out


def _bitonic_argsort_kernel(h_ref, idx_ref):
    v = h_ref[...]
    ix = jax.lax.broadcasted_iota(jnp.int32, v.shape, 1)
    rb, n = v.shape
    pos = jax.lax.broadcasted_iota(jnp.int32, v.shape, 1)
    k = 2
    while k <= n:
        j = k // 2
        asc = (pos & k) == 0
        while j >= 1:
            bitc = (pos & j) == 0
            vm = jnp.roll(v, -j, axis=1)
            vp = jnp.roll(v, j, axis=1)
            im = jnp.roll(ix, -j, axis=1)
            ip = jnp.roll(ix, j, axis=1)
            pv = jnp.where(bitc, vm, vp)
            pi = jnp.where(bitc, im, ip)
            less = (v < pv) | ((v == pv) & (ix < pi))
            keep = less ^ bitc ^ asc
            v = jnp.where(keep, v, pv)
            ix = jnp.where(keep, ix, pi)
            j //= 2
        k *= 2
    idx_ref[...] = ix


def _tc_argsort(h, rows_per_block=32):
    shape = h.shape
    n = shape[-1]
    rows = 1
    for s in shape[:-1]:
        rows *= s
    rows_per_block = min(rows_per_block, rows)
    h2 = h.reshape(rows, n)
    idx = pl.pallas_call(
        _bitonic_argsort_kernel,
        grid=(rows // rows_per_block,),
        in_specs=[pl.BlockSpec((rows_per_block, n), lambda i: (i, 0))],
        out_specs=pl.BlockSpec((rows_per_block, n), lambda i: (i, 0)),
        out_shape=jax.ShapeDtypeStruct((rows, n), jnp.int32),
    )(h2)
    return idx.reshape(shape)


_N_CHUNKS = 2


def _hash_values(X, n_hashes, bs, r, rng):
    hb, seqlen, dim = X.shape
    ka, kb = jax.random.split(rng)
    alpha = jax.random.normal(ka, (n_hashes, dim), dtype=X.dtype)
    beta = jax.random.uniform(kb, (n_hashes, 1, 1), dtype=X.dtype) * r
    Xh = X.reshape(n_hashes, bs, seqlen, dim)
    proj = jnp.einsum('hbsd,hd->hbs', Xh, alpha)
    return (proj + beta) / r


def kernel(queries, keys, values):
    n_hashes = _N_HASHES
    bs, q_seqlen, dim = queries.shape
    k_seqlen = keys.shape[1]
    v_dim = values.shape[-1]

    Qx, Kx = _xbox_plus(queries, keys)
    Qx = jnp.tile(Qx, (n_hashes, 1, 1))
    Kx = jnp.tile(Kx, (n_hashes, 1, 1))
    rng = jax.random.key(42)
    rq, rk = jax.random.split(rng)
    q_hashes = _hash_values(Qx, n_hashes, bs, _LSH_R, rq)
    k_hashes = _hash_values(Kx, n_hashes, bs, _LSH_R, rk)

    boff = (jnp.arange(bs, dtype=jnp.int32) * q_seqlen)[None, :, None]

    qtab = jnp.concatenate(
        [queries, jnp.zeros((bs, q_seqlen, _ROW - dim), queries.dtype)],
        axis=-1).reshape(bs * q_seqlen, _ROW)
    kvtab = jnp.concatenate([keys, values], axis=-1).reshape(
        bs * k_seqlen, _ROW)

    hc = n_hashes // _N_CHUNKS
    hoff = (jnp.arange(hc, dtype=jnp.int32) * (bs * q_seqlen))[:, None, None]
    Bc = hc * bs * q_seqlen
    nblk = q_seqlen // _BLK
    o_uns_chunks = []
    for c in range(_N_CHUNKS):
        qp = _tc_argsort(q_hashes[c * hc:(c + 1) * hc])
        kp = _tc_argsort(k_hashes[c * hc:(c + 1) * hc])
        q_abs = qp.astype(jnp.int32) + boff
        k_abs = kp.astype(jnp.int32) + boff
        s_abs = q_abs + hoff
        sq, skv = _sc_gather2(qtab, kvtab, q_abs.reshape(1, Bc),
                              k_abs.reshape(1, Bc))
        bo_ext = _block_attention(sq, skv, hc * bs, nblk, dim, v_dim)
        o_uns = _sc_scatter(bo_ext.reshape(Bc, _ROW), s_abs.reshape(1, Bc),
                            Bc)
        o_uns_chunks.append(o_uns.reshape(hc, bs, q_seqlen, _ROW))

    out = _combine(o_uns_chunks, bs, q_seqlen, v_dim)
    return out
